# Initial kernel scaffold; baseline (speedup 1.0000x reference)
import jax
import jax.numpy as jnp
from jax.experimental import pallas as pl


def kernel(transformation, src_points, tar_points, covs_src, covs_tar):
    raise NotImplementedError("write your pallas kernel here")


# speedup vs baseline: 23.2592x; 23.2592x over previous
import functools

import jax
import jax.numpy as jnp
from jax import lax
from jax.experimental import pallas as pl
from jax.experimental.pallas import tpu as pltpu
from jax.experimental.pallas import tpu_sc as plsc

N_SRC = 4000
N_TAR = 10000
BATCH = 500
BS = 1000
TCH = 2048
TPAD = 10240
NCH = TPAD // TCH
B_PAD = 4096


def _nn_body(src_ref, tarT_ref, Tt_ref, out_ref):
    s = jnp.dot(src_ref[...], Tt_ref[...], preferred_element_type=jnp.float32)
    s2 = jnp.sum(s * s, axis=1, keepdims=True)
    best_v = jnp.full((BS, 1), jnp.float32(jnp.inf), dtype=jnp.float32)
    best_i = jnp.zeros((BS, 1), dtype=jnp.int32)
    for c in range(NCH):
        tc = tarT_ref[:, c * TCH:(c + 1) * TCH]
        t2 = jnp.sum(tc * tc, axis=0, keepdims=True)
        d2 = (s2 + t2) - 2.0 * jnp.dot(s, tc, preferred_element_type=jnp.float32)
        d = jnp.sqrt(jnp.maximum(d2, 0.0))
        cv = jnp.min(d, axis=1, keepdims=True)
        ii = lax.broadcasted_iota(jnp.int32, (BS, TCH), 1)
        ci = jnp.min(jnp.where(d == cv, ii, jnp.int32(0x7FFFFFFF)),
                     axis=1, keepdims=True) + c * TCH
        upd = cv < best_v
        best_v = jnp.where(upd, cv, best_v)
        best_i = jnp.where(upd, ci, best_i)
    out_ref[0, :, :] = best_i


def _nn_indices(src, tarT_pad, Tt):
    out = pl.pallas_call(
        _nn_body,
        grid=(N_SRC // BS,),
        in_specs=[
            pl.BlockSpec((BS, 4), lambda i: (i, 0)),
            pl.BlockSpec((4, TPAD), lambda i: (0, 0)),
            pl.BlockSpec((4, 4), lambda i: (0, 0)),
        ],
        out_specs=pl.BlockSpec((1, BS, 1), lambda i: (i, 0, 0)),
        out_shape=jax.ShapeDtypeStruct((N_SRC // BS, BS, 1), jnp.int32),
    )(src, tarT_pad, Tt)
    return out.reshape(N_SRC)


def _gather_rows(table, idx_pad):
    nw = 32
    bpw = B_PAD // nw
    dcols = table.shape[1]
    mesh = plsc.VectorSubcoreMesh(core_axis_name="c", subcore_axis_name="s")

    @functools.partial(
        pl.kernel,
        mesh=mesh,
        out_type=jax.ShapeDtypeStruct((B_PAD, dcols), jnp.float32),
        scratch_types=[
            pltpu.VMEM((bpw,), jnp.int32),
            pltpu.VMEM((bpw, dcols), jnp.float32),
            pltpu.SemaphoreType.DMA,
        ],
    )
    def k(table_hbm, idx_hbm, out_hbm, idx_v, rows_v, sem):
        wid = lax.axis_index("s") * 2 + lax.axis_index("c")
        base = wid * bpw
        pltpu.sync_copy(idx_hbm.at[pl.ds(base, bpw)], idx_v)
        pltpu.async_copy(table_hbm.at[idx_v], rows_v, sem).wait()
        pltpu.sync_copy(rows_v, out_hbm.at[pl.ds(base, bpw)])

    return k(table, idx_pad)


def _loss_body(gT_ref, csT_ref, srcT_ref, Tt_ref, kron_ref, out_ref):
    gT = gT_ref[...]
    tcs = jnp.dot(kron_ref[...], csT_ref[...],
                  preferred_element_type=jnp.float32)
    pt = jnp.dot(Tt_ref[...], srcT_ref[...],
                 preferred_element_type=jnp.float32)
    r0 = gT[16:17, :] - pt[0:1, :]
    r1 = gT[17:18, :] - pt[1:2, :]
    r2 = gT[18:19, :] - pt[2:3, :]

    def m(a, b):
        k = 4 * a + b
        return gT[k:k + 1, :] + tcs[k:k + 1, :]

    a_, b_, c_ = m(0, 0), m(0, 1), m(0, 2)
    d_, e_, f_ = m(1, 0), m(1, 1), m(1, 2)
    g_, h_, i_ = m(2, 0), m(2, 1), m(2, 2)
    A = e_ * i_ - f_ * h_
    B = c_ * h_ - b_ * i_
    C = b_ * f_ - c_ * e_
    D = f_ * g_ - d_ * i_
    E = a_ * i_ - c_ * g_
    F = c_ * d_ - a_ * f_
    G = d_ * h_ - e_ * g_
    H = b_ * g_ - a_ * h_
    I = a_ * e_ - b_ * d_
    det = a_ * A + b_ * D + c_ * G
    idet = 1.0 / det
    v0 = (r0 * A + r1 * D + r2 * G) * idet
    v1 = (r0 * B + r1 * E + r2 * H) * idet
    v2 = (r0 * C + r1 * F + r2 * I) * idet
    P = jnp.concatenate([v0, v1, v2, r0, r1, r2], axis=0)
    row = lax.broadcasted_iota(jnp.int32, (N_SRC, N_SRC // BATCH), 0) // BATCH
    col = lax.broadcasted_iota(jnp.int32, (N_SRC, N_SRC // BATCH), 1)
    Bm = (row == col).astype(jnp.float32)
    S = jnp.dot(P, Bm, preferred_element_type=jnp.float32)
    nb = N_SRC // BATCH
    prod = S[0:3, :] * S[3:6, :]
    tot = jnp.sum(jnp.sum(prod, axis=1, keepdims=True), axis=0, keepdims=True)
    out_ref[...] = tot * (1.0 / (float(BATCH) * float(BATCH) * float(nb)))


def _loss(gT, csT, srcT, Tt, kron):
    out = pl.pallas_call(
        _loss_body,
        in_specs=[pl.BlockSpec(gT.shape, lambda: (0, 0)),
                  pl.BlockSpec(csT.shape, lambda: (0, 0)),
                  pl.BlockSpec(srcT.shape, lambda: (0, 0)),
                  pl.BlockSpec((4, 4), lambda: (0, 0)),
                  pl.BlockSpec((16, 16), lambda: (0, 0))],
        out_specs=pl.BlockSpec((1, 1), lambda: (0, 0)),
        out_shape=jax.ShapeDtypeStruct((1, 1), jnp.float32),
    )(gT, csT, srcT, Tt, kron)
    return out[0, 0]


def kernel(transformation, src_points, tar_points, covs_src, covs_tar):
    T = transformation
    Tt = T.T
    kron = jnp.kron(T, T)
    tarT_pad = jnp.concatenate(
        [tar_points.T,
         jnp.full((4, TPAD - N_TAR), 1e6, dtype=jnp.float32)], axis=1)
    indices = _nn_indices(src_points, tarT_pad, Tt)
    idx_pad = jnp.concatenate(
        [indices, jnp.zeros((B_PAD - N_SRC,), dtype=jnp.int32)])
    table = jnp.concatenate(
        [covs_tar.reshape(N_TAR, 16), tar_points,
         jnp.zeros((N_TAR, 108), dtype=jnp.float32)], axis=1)
    gathered = _gather_rows(table, idx_pad)
    gT = gathered[:N_SRC, :32].T
    csT = covs_src.reshape(N_SRC, 16).T
    srcT = src_points.T
    return _loss(gT, csT, srcT, Tt, kron)


# speedup vs baseline: 24.5505x; 1.0555x over previous
import functools

import jax
import jax.numpy as jnp
from jax import lax
from jax.experimental import pallas as pl
from jax.experimental.pallas import tpu as pltpu
from jax.experimental.pallas import tpu_sc as plsc

N_SRC = 4000
N_TAR = 10000
BATCH = 500
BS = 1000
TCH = 2048
TPAD = 10240
NCH = TPAD // TCH
B_PAD = 4096


def _nn_body(src_ref, tarT_ref, Tt_ref, out_ref):
    s = jnp.dot(src_ref[...], Tt_ref[...], preferred_element_type=jnp.float32)
    s2 = jnp.sum(s * s, axis=1, keepdims=True)
    sm2 = -2.0 * s
    runv = jnp.full((BS, TCH), jnp.float32(jnp.inf), dtype=jnp.float32)
    runc = jnp.zeros((BS, TCH), dtype=jnp.int32)
    for c in range(NCH):
        tc = tarT_ref[:, c * TCH:(c + 1) * TCH]
        t2 = jnp.sum(tc * tc, axis=0, keepdims=True)
        d2 = (s2 + t2) + jnp.dot(sm2, tc, preferred_element_type=jnp.float32)
        d = jnp.sqrt(jnp.maximum(d2, 0.0))
        upd = d < runv
        runv = jnp.where(upd, d, runv)
        runc = jnp.where(upd, c, runc)
    m = jnp.min(runv, axis=1, keepdims=True)
    ii = lax.broadcasted_iota(jnp.int32, (BS, TCH), 1)
    jfull = runc * TCH + ii
    best_i = jnp.min(jnp.where(runv == m, jfull, jnp.int32(0x7FFFFFFF)),
                     axis=1, keepdims=True)
    out_ref[0, :, :] = best_i


def _nn_indices(src, tarT_pad, Tt):
    out = pl.pallas_call(
        _nn_body,
        grid=(N_SRC // BS,),
        in_specs=[
            pl.BlockSpec((BS, 4), lambda i: (i, 0)),
            pl.BlockSpec((4, TPAD), lambda i: (0, 0)),
            pl.BlockSpec((4, 4), lambda i: (0, 0)),
        ],
        out_specs=pl.BlockSpec((1, BS, 1), lambda i: (i, 0, 0)),
        out_shape=jax.ShapeDtypeStruct((N_SRC // BS, BS, 1), jnp.int32),
    )(src, tarT_pad, Tt)
    return out.reshape(N_SRC)


def _gather_rows(table, idx_pad):
    nw = 32
    bpw = B_PAD // nw
    dcols = table.shape[1]
    mesh = plsc.VectorSubcoreMesh(core_axis_name="c", subcore_axis_name="s")

    @functools.partial(
        pl.kernel,
        mesh=mesh,
        out_type=jax.ShapeDtypeStruct((B_PAD, dcols), jnp.float32),
        scratch_types=[
            pltpu.VMEM((bpw,), jnp.int32),
            pltpu.VMEM((bpw, dcols), jnp.float32),
            pltpu.SemaphoreType.DMA,
        ],
    )
    def k(table_hbm, idx_hbm, out_hbm, idx_v, rows_v, sem):
        wid = lax.axis_index("s") * 2 + lax.axis_index("c")
        base = wid * bpw
        pltpu.sync_copy(idx_hbm.at[pl.ds(base, bpw)], idx_v)
        pltpu.async_copy(table_hbm.at[idx_v], rows_v, sem).wait()
        pltpu.sync_copy(rows_v, out_hbm.at[pl.ds(base, bpw)])

    return k(table, idx_pad)


def _loss_body(gT_ref, csT_ref, srcT_ref, Tt_ref, kron_ref, out_ref):
    gT = gT_ref[...]
    tcs = jnp.dot(kron_ref[...], csT_ref[...],
                  preferred_element_type=jnp.float32)
    pt = jnp.dot(Tt_ref[...], srcT_ref[...],
                 preferred_element_type=jnp.float32)
    r0 = gT[16:17, :] - pt[0:1, :]
    r1 = gT[17:18, :] - pt[1:2, :]
    r2 = gT[18:19, :] - pt[2:3, :]

    def m(a, b):
        k = 4 * a + b
        return gT[k:k + 1, :] + tcs[k:k + 1, :]

    a_, b_, c_ = m(0, 0), m(0, 1), m(0, 2)
    d_, e_, f_ = m(1, 0), m(1, 1), m(1, 2)
    g_, h_, i_ = m(2, 0), m(2, 1), m(2, 2)
    A = e_ * i_ - f_ * h_
    B = c_ * h_ - b_ * i_
    C = b_ * f_ - c_ * e_
    D = f_ * g_ - d_ * i_
    E = a_ * i_ - c_ * g_
    F = c_ * d_ - a_ * f_
    G = d_ * h_ - e_ * g_
    H = b_ * g_ - a_ * h_
    I = a_ * e_ - b_ * d_
    det = a_ * A + b_ * D + c_ * G
    idet = 1.0 / det
    v0 = (r0 * A + r1 * D + r2 * G) * idet
    v1 = (r0 * B + r1 * E + r2 * H) * idet
    v2 = (r0 * C + r1 * F + r2 * I) * idet
    P = jnp.concatenate([v0, v1, v2, r0, r1, r2], axis=0)
    row = lax.broadcasted_iota(jnp.int32, (N_SRC, N_SRC // BATCH), 0) // BATCH
    col = lax.broadcasted_iota(jnp.int32, (N_SRC, N_SRC // BATCH), 1)
    Bm = (row == col).astype(jnp.float32)
    S = jnp.dot(P, Bm, preferred_element_type=jnp.float32)
    nb = N_SRC // BATCH
    prod = S[0:3, :] * S[3:6, :]
    tot = jnp.sum(jnp.sum(prod, axis=1, keepdims=True), axis=0, keepdims=True)
    out_ref[...] = tot * (1.0 / (float(BATCH) * float(BATCH) * float(nb)))


def _loss(gT, csT, srcT, Tt, kron):
    out = pl.pallas_call(
        _loss_body,
        in_specs=[pl.BlockSpec(gT.shape, lambda: (0, 0)),
                  pl.BlockSpec(csT.shape, lambda: (0, 0)),
                  pl.BlockSpec(srcT.shape, lambda: (0, 0)),
                  pl.BlockSpec((4, 4), lambda: (0, 0)),
                  pl.BlockSpec((16, 16), lambda: (0, 0))],
        out_specs=pl.BlockSpec((1, 1), lambda: (0, 0)),
        out_shape=jax.ShapeDtypeStruct((1, 1), jnp.float32),
    )(gT, csT, srcT, Tt, kron)
    return out[0, 0]


def kernel(transformation, src_points, tar_points, covs_src, covs_tar):
    T = transformation
    Tt = T.T
    kron = jnp.kron(T, T)
    tarT_pad = jnp.concatenate(
        [tar_points.T,
         jnp.full((4, TPAD - N_TAR), 1e6, dtype=jnp.float32)], axis=1)
    indices = _nn_indices(src_points, tarT_pad, Tt)
    idx_pad = jnp.concatenate(
        [indices, jnp.zeros((B_PAD - N_SRC,), dtype=jnp.int32)])
    table = jnp.concatenate(
        [covs_tar.reshape(N_TAR, 16), tar_points,
         jnp.zeros((N_TAR, 108), dtype=jnp.float32)], axis=1)
    gathered = _gather_rows(table, idx_pad)
    gT = gathered[:N_SRC, :32].T
    csT = covs_src.reshape(N_SRC, 16).T
    srcT = src_points.T
    return _loss(gT, csT, srcT, Tt, kron)


# speedup vs baseline: 30.9038x; 1.2588x over previous
import functools

import jax
import jax.numpy as jnp
from jax import lax
from jax.experimental import pallas as pl
from jax.experimental.pallas import tpu as pltpu
from jax.experimental.pallas import tpu_sc as plsc

N_SRC = 4000
N_TAR = 10000
BATCH = 500
BS = 1000
TCH = 2048
TPAD = 10240
NCH = TPAD // TCH
B_PAD = 4096


def _nn_body(src_ref, tarT_ref, Tt_ref, out_ref):
    s = jnp.dot(src_ref[...], Tt_ref[...], preferred_element_type=jnp.float32)
    s2 = jnp.sum(s * s, axis=1, keepdims=True)
    sm2 = -2.0 * s
    runv = jnp.full((BS, TCH), jnp.float32(jnp.inf), dtype=jnp.float32)
    runc = jnp.zeros((BS, TCH), dtype=jnp.int32)
    for c in range(NCH):
        tc = tarT_ref[:, c * TCH:(c + 1) * TCH]
        t2 = jnp.sum(tc * tc, axis=0, keepdims=True)
        d2 = (s2 + t2) + jnp.dot(sm2, tc, preferred_element_type=jnp.float32)
        upd = d2 < runv
        runv = jnp.where(upd, d2, runv)
        runc = jnp.where(upd, c, runc)
    m2 = jnp.min(runv, axis=1, keepdims=True)
    z = jnp.sqrt(jnp.maximum(m2, 0.0))
    pb = lax.bitcast_convert_type(z * z, jnp.int32)
    hi = m2
    for k in range(-3, 5):
        yk = lax.bitcast_convert_type(pb + k, jnp.float32)
        hi = jnp.where(jnp.sqrt(yk) == z, yk, hi)
    ii = lax.broadcasted_iota(jnp.int32, (BS, TCH), 1)
    jfull = runc * TCH + ii
    best_i = jnp.min(jnp.where(runv <= hi, jfull, jnp.int32(0x7FFFFFFF)),
                     axis=1, keepdims=True)
    out_ref[0, :, :] = best_i


def _nn_indices(src, tarT_pad, Tt):
    out = pl.pallas_call(
        _nn_body,
        grid=(N_SRC // BS,),
        in_specs=[
            pl.BlockSpec((BS, 4), lambda i: (i, 0)),
            pl.BlockSpec((4, TPAD), lambda i: (0, 0)),
            pl.BlockSpec((4, 4), lambda i: (0, 0)),
        ],
        out_specs=pl.BlockSpec((1, BS, 1), lambda i: (i, 0, 0)),
        out_shape=jax.ShapeDtypeStruct((N_SRC // BS, BS, 1), jnp.int32),
    )(src, tarT_pad, Tt)
    return out.reshape(N_SRC)


def _gather_rows(table, idx_pad):
    nw = 32
    bpw = B_PAD // nw
    dcols = table.shape[1]
    mesh = plsc.VectorSubcoreMesh(core_axis_name="c", subcore_axis_name="s")

    @functools.partial(
        pl.kernel,
        mesh=mesh,
        out_type=jax.ShapeDtypeStruct((B_PAD, dcols), jnp.float32),
        scratch_types=[
            pltpu.VMEM((bpw,), jnp.int32),
            pltpu.VMEM((bpw, dcols), jnp.float32),
            pltpu.SemaphoreType.DMA,
        ],
    )
    def k(table_hbm, idx_hbm, out_hbm, idx_v, rows_v, sem):
        wid = lax.axis_index("s") * 2 + lax.axis_index("c")
        base = wid * bpw
        pltpu.sync_copy(idx_hbm.at[pl.ds(base, bpw)], idx_v)
        pltpu.async_copy(table_hbm.at[idx_v], rows_v, sem).wait()
        pltpu.sync_copy(rows_v, out_hbm.at[pl.ds(base, bpw)])

    return k(table, idx_pad)


def _loss_body(gT_ref, csT_ref, srcT_ref, Tt_ref, kron_ref, out_ref):
    gT = gT_ref[...]
    tcs = jnp.dot(kron_ref[...], csT_ref[...],
                  preferred_element_type=jnp.float32)
    pt = jnp.dot(Tt_ref[...], srcT_ref[...],
                 preferred_element_type=jnp.float32)
    r0 = gT[16:17, :] - pt[0:1, :]
    r1 = gT[17:18, :] - pt[1:2, :]
    r2 = gT[18:19, :] - pt[2:3, :]

    def m(a, b):
        k = 4 * a + b
        return gT[k:k + 1, :] + tcs[k:k + 1, :]

    a_, b_, c_ = m(0, 0), m(0, 1), m(0, 2)
    d_, e_, f_ = m(1, 0), m(1, 1), m(1, 2)
    g_, h_, i_ = m(2, 0), m(2, 1), m(2, 2)
    A = e_ * i_ - f_ * h_
    B = c_ * h_ - b_ * i_
    C = b_ * f_ - c_ * e_
    D = f_ * g_ - d_ * i_
    E = a_ * i_ - c_ * g_
    F = c_ * d_ - a_ * f_
    G = d_ * h_ - e_ * g_
    H = b_ * g_ - a_ * h_
    I = a_ * e_ - b_ * d_
    det = a_ * A + b_ * D + c_ * G
    idet = 1.0 / det
    v0 = (r0 * A + r1 * D + r2 * G) * idet
    v1 = (r0 * B + r1 * E + r2 * H) * idet
    v2 = (r0 * C + r1 * F + r2 * I) * idet
    P = jnp.concatenate([v0, v1, v2, r0, r1, r2], axis=0)
    row = lax.broadcasted_iota(jnp.int32, (N_SRC, N_SRC // BATCH), 0) // BATCH
    col = lax.broadcasted_iota(jnp.int32, (N_SRC, N_SRC // BATCH), 1)
    Bm = (row == col).astype(jnp.float32)
    S = jnp.dot(P, Bm, preferred_element_type=jnp.float32)
    nb = N_SRC // BATCH
    prod = S[0:3, :] * S[3:6, :]
    tot = jnp.sum(jnp.sum(prod, axis=1, keepdims=True), axis=0, keepdims=True)
    out_ref[...] = tot * (1.0 / (float(BATCH) * float(BATCH) * float(nb)))


def _loss(gT, csT, srcT, Tt, kron):
    out = pl.pallas_call(
        _loss_body,
        in_specs=[pl.BlockSpec(gT.shape, lambda: (0, 0)),
                  pl.BlockSpec(csT.shape, lambda: (0, 0)),
                  pl.BlockSpec(srcT.shape, lambda: (0, 0)),
                  pl.BlockSpec((4, 4), lambda: (0, 0)),
                  pl.BlockSpec((16, 16), lambda: (0, 0))],
        out_specs=pl.BlockSpec((1, 1), lambda: (0, 0)),
        out_shape=jax.ShapeDtypeStruct((1, 1), jnp.float32),
    )(gT, csT, srcT, Tt, kron)
    return out[0, 0]


def kernel(transformation, src_points, tar_points, covs_src, covs_tar):
    T = transformation
    Tt = T.T
    kron = jnp.kron(T, T)
    tarT_pad = jnp.concatenate(
        [tar_points.T,
         jnp.full((4, TPAD - N_TAR), 1e6, dtype=jnp.float32)], axis=1)
    indices = _nn_indices(src_points, tarT_pad, Tt)
    idx_pad = jnp.concatenate(
        [indices, jnp.zeros((B_PAD - N_SRC,), dtype=jnp.int32)])
    table = jnp.concatenate(
        [covs_tar.reshape(N_TAR, 16), tar_points,
         jnp.zeros((N_TAR, 108), dtype=jnp.float32)], axis=1)
    gathered = _gather_rows(table, idx_pad)
    gT = gathered[:N_SRC, :32].T
    csT = covs_src.reshape(N_SRC, 16).T
    srcT = src_points.T
    return _loss(gT, csT, srcT, Tt, kron)
